# baseline (speedup 1.0000x reference)
import functools

import jax
import jax.numpy as jnp
from jax import lax
from jax.experimental import pallas as pl
from jax.experimental.pallas import tpu as pltpu
from jax.experimental.pallas import tpu_sc as plsc

_NW = 32
_RC = 64
_CHUNK = _RC * 255


def _sc_body(vals_hbm, binsp_hbm, b0_hbm, bm_hbm, out_hbm,
             vals_v, bins_v, b0_v, bm_v,
             rowbuf0, rowbuf1, colbuf0, colbuf1, sem0, sem1,
             *, rows_per_w):
    nchunks = rows_per_w // _RC
    wid = lax.axis_index("s") * 2 + lax.axis_index("c")
    base = wid * rows_per_w

    pltpu.sync_copy(vals_hbm.at[pl.ds(base, rows_per_w)], vals_v)
    pltpu.sync_copy(binsp_hbm, bins_v)
    pltpu.sync_copy(b0_hbm, b0_v)
    pltpu.sync_copy(bm_hbm, bm_v)

    zf = jnp.zeros((16,), jnp.float32)
    zi = jnp.zeros((16,), jnp.int32)

    def zero_buf(k, _):
        rowbuf0[pl.ds(k * 16, 16)] = zf
        rowbuf1[pl.ds(k * 16, 16)] = zf
        return 0

    lax.fori_loop(0, _CHUNK // 16, zero_buf, 0)
    for t in range(8):
        colbuf0[pl.ds(t * 16, 16)] = zi
        colbuf1[pl.ds(t * 16, 16)] = zi

    b0 = b0_v[pl.ds(0, 16)]
    bm = bm_v[pl.ds(0, 16)]
    lane = lax.iota(jnp.int32, 16)

    def do_chunk(c, rbuf, cbuf, sem):
        @pl.when(c >= 2)
        def _wait_prior():
            pltpu.make_async_copy(
                rbuf, out_hbm.at[pl.ds(0, _CHUNK)], sem).wait()

        for t in range(8):
            stale = cbuf[pl.ds(t * 16, 16)]
            plsc.store_scatter(rbuf, [stale], zf)

        for t in range(4):
            v = vals_v[pl.ds(c * _RC + t * 16, 16)]
            v = jnp.minimum(jnp.maximum(v, b0), bm)
            idx = zi
            for step in (128, 64, 32, 16, 8, 4, 2, 1):
                cand = idx + step
                g = plsc.load_gather(bins_v, [cand - 1])
                idx = jnp.where(g < v, cand, idx)
            left = jnp.maximum(idx - 1, 0)
            lv = plsc.load_gather(bins_v, [left])
            rv = plsc.load_gather(bins_v, [left + 1])
            rw = (v - lv) / (rv - lv)
            lw = 1.0 - rw
            posl = (t * 16 + lane) * 255 + left
            posr = posl + 1
            plsc.store_scatter(rbuf, [posl], lw)
            plsc.store_scatter(rbuf, [posr], rw)
            cbuf[pl.ds(t * 32, 16)] = posl
            cbuf[pl.ds(t * 32 + 16, 16)] = posr

        pltpu.make_async_copy(
            rbuf, out_hbm.at[pl.ds((base + c * _RC) * 255, _CHUNK)],
            sem).start()

    def chunk_pair(p, _):
        do_chunk(2 * p, rowbuf0, colbuf0, sem0)
        do_chunk(2 * p + 1, rowbuf1, colbuf1, sem1)
        return 0

    lax.fori_loop(0, nchunks // 2, chunk_pair, 0)
    pltpu.make_async_copy(rowbuf0, out_hbm.at[pl.ds(0, _CHUNK)], sem0).wait()
    pltpu.make_async_copy(rowbuf1, out_hbm.at[pl.ds(0, _CHUNK)], sem1).wait()


def kernel(values, bin_values):
    r0, r1 = values.shape
    nbins = bin_values.shape[0]
    n = r0 * r1
    rows_per_w = n // _NW
    flat = values.reshape(n)
    binsp = jnp.concatenate([bin_values, bin_values[-1:]])
    b0 = jnp.broadcast_to(bin_values[0], (16,))
    bm = jnp.broadcast_to(bin_values[-1], (16,))

    mesh = plsc.VectorSubcoreMesh(core_axis_name="c", subcore_axis_name="s")
    run = pl.kernel(
        functools.partial(_sc_body, rows_per_w=rows_per_w),
        out_type=jax.ShapeDtypeStruct((n * nbins,), jnp.float32),
        mesh=mesh,
        scratch_types=[
            pltpu.VMEM((rows_per_w,), jnp.float32),
            pltpu.VMEM((nbins + 1,), jnp.float32),
            pltpu.VMEM((16,), jnp.float32),
            pltpu.VMEM((16,), jnp.float32),
            pltpu.VMEM((_CHUNK,), jnp.float32),
            pltpu.VMEM((_CHUNK,), jnp.float32),
            pltpu.VMEM((128,), jnp.int32),
            pltpu.VMEM((128,), jnp.int32),
            pltpu.SemaphoreType.DMA,
            pltpu.SemaphoreType.DMA,
        ],
        compiler_params=pltpu.CompilerParams(
            needs_layout_passes=False,
        ),
    )
    out = run(flat, binsp, b0, bm)
    return out.reshape(r0, r1, nbins)


# speedup vs baseline: 3.6015x; 3.6015x over previous
import functools

import jax
import jax.numpy as jnp
from jax import lax
from jax.experimental import pallas as pl
from jax.experimental.pallas import tpu as pltpu
from jax.experimental.pallas import tpu_sc as plsc

_NW = 32
_SL = 4
_RC = _SL * 50
_NV = _RC // 16 + 1


def _sc_body(vals_hbm, binsp_hbm, b0_hbm, bm_hbm, zeros_hbm, out_hbm,
             vals_v, bins_v, b0_v, bm_v,
             rowbuf0, rowbuf1, colbuf0, colbuf1, sem0, sem1,
             *, slabs_per_w, r1, nbins):
    nchunks = slabs_per_w // _SL
    wid = lax.axis_index("s") * 2 + lax.axis_index("c")
    slab0 = wid * slabs_per_w

    pltpu.sync_copy(vals_hbm.at[pl.ds(slab0 * r1, slabs_per_w * r1)],
                    vals_v.at[pl.ds(0, slabs_per_w * r1)])
    pltpu.sync_copy(binsp_hbm, bins_v)
    pltpu.sync_copy(b0_hbm, b0_v)
    pltpu.sync_copy(bm_hbm, bm_v)
    pltpu.sync_copy(zeros_hbm, rowbuf0)
    pltpu.sync_copy(zeros_hbm, rowbuf1)

    zf = jnp.zeros((16,), jnp.float32)
    zi = jnp.zeros((16,), jnp.int32)
    for t in range(_NV):
        colbuf0[pl.ds(t * 16, 16)] = zi
        colbuf1[pl.ds(t * 16, 16)] = zi

    b0 = b0_v[pl.ds(0, 16)]
    bm = bm_v[pl.ds(0, 16)]
    lane = lax.iota(jnp.int32, 16)
    full = lane < 16

    def do_chunk(c, rbuf, cbuf, sem):
        @pl.when(c >= 2)
        def _wait_prior():
            pltpu.make_async_copy(
                rbuf, out_hbm.at[pl.ds(0, _SL)], sem).wait()

        for t in range(_NV):
            rr = t * 16 + lane
            mask = full if t < _NV - 1 else (lane < (_RC - (_NV - 1) * 16))
            slab_l = rr // r1
            row_l = rr - slab_l * r1
            stale = cbuf[pl.ds(t * 16, 16)]
            plsc.store_scatter(rbuf, [slab_l, row_l, stale], zf, mask=mask)
            plsc.store_scatter(rbuf, [slab_l, row_l, stale + 1], zf, mask=mask)

            v = vals_v[pl.ds(c * _RC + t * 16, 16)]
            v = jnp.minimum(jnp.maximum(v, b0), bm)
            idx = zi
            for step in (128, 64, 32, 16, 8, 4, 2, 1):
                cand = idx + step
                g = plsc.load_gather(bins_v, [cand - 1])
                idx = jnp.where(g < v, cand, idx)
            left = jnp.maximum(idx - 1, 0)
            lv = plsc.load_gather(bins_v, [left])
            rv = plsc.load_gather(bins_v, [left + 1])
            rw = (v - lv) / (rv - lv)
            lw = 1.0 - rw
            plsc.store_scatter(rbuf, [slab_l, row_l, left], lw, mask=mask)
            plsc.store_scatter(rbuf, [slab_l, row_l, left + 1], rw, mask=mask)
            cbuf[pl.ds(t * 16, 16)] = left

        pltpu.make_async_copy(
            rbuf, out_hbm.at[pl.ds(slab0 + c * _SL, _SL)], sem).start()

    def chunk_pair(p, _):
        do_chunk(2 * p, rowbuf0, colbuf0, sem0)
        do_chunk(2 * p + 1, rowbuf1, colbuf1, sem1)
        return 0

    lax.fori_loop(0, nchunks // 2, chunk_pair, 0)
    pltpu.make_async_copy(rowbuf0, out_hbm.at[pl.ds(0, _SL)], sem0).wait()
    pltpu.make_async_copy(rowbuf1, out_hbm.at[pl.ds(0, _SL)], sem1).wait()


def kernel(values, bin_values):
    r0, r1 = values.shape
    nbins = bin_values.shape[0]
    n = r0 * r1
    slabs_per_w = r0 // _NW
    flat = values.reshape(n)
    binsp = jnp.concatenate([bin_values, bin_values[-1:]])
    b0 = jnp.broadcast_to(bin_values[0], (16,))
    bm = jnp.broadcast_to(bin_values[-1], (16,))
    zeros = jnp.zeros((_SL, r1, nbins), jnp.float32)

    mesh = plsc.VectorSubcoreMesh(core_axis_name="c", subcore_axis_name="s")
    run = pl.kernel(
        functools.partial(_sc_body, slabs_per_w=slabs_per_w, r1=r1,
                          nbins=nbins),
        out_type=jax.ShapeDtypeStruct((r0, r1, nbins), jnp.float32),
        mesh=mesh,
        scratch_types=[
            pltpu.VMEM((slabs_per_w * r1 + 16,), jnp.float32),
            pltpu.VMEM((nbins + 1,), jnp.float32),
            pltpu.VMEM((16,), jnp.float32),
            pltpu.VMEM((16,), jnp.float32),
            pltpu.VMEM((_SL, r1, nbins), jnp.float32),
            pltpu.VMEM((_SL, r1, nbins), jnp.float32),
            pltpu.VMEM((_NV * 16,), jnp.int32),
            pltpu.VMEM((_NV * 16,), jnp.int32),
            pltpu.SemaphoreType.DMA,
            pltpu.SemaphoreType.DMA,
        ],
        compiler_params=pltpu.CompilerParams(
            needs_layout_passes=False,
        ),
    )
    out = run(flat, binsp, b0, bm, zeros)
    return out


# speedup vs baseline: 5.6188x; 1.5601x over previous
import functools

import jax
import jax.numpy as jnp
from jax.experimental import pallas as pl
from jax.experimental.pallas import tpu as pltpu


def _twohot_body(v_ref, b_ref, u1_ref, au1_ref, u2_ref, au2_ref, o_ref):
    v = jnp.maximum(v_ref[...], b_ref[0, 0, 0])
    t1 = jnp.clip(v * u1_ref[...] - au1_ref[...], 0.0, 1.0)
    t2 = jnp.clip(v * u2_ref[...] - au2_ref[...], 0.0, 1.0)
    o_ref[...] = t1 - t2


def kernel(values, bin_values):
    r0, r1 = values.shape
    nbins = bin_values.shape[0]
    bins = bin_values
    u1i = 1.0 / (bins[1:] - bins[:-1])
    u1 = jnp.concatenate([jnp.zeros((1,), jnp.float32), u1i])
    au1 = jnp.concatenate([jnp.full((1,), -1.0, jnp.float32), bins[:-1] * u1i])
    nxt = jnp.concatenate([bins[1:], bins[-1:]])
    d2 = nxt - bins
    u2 = jnp.where(d2 > 0, 1.0 / jnp.maximum(d2, 1e-30), 0.0)
    au2 = bins * u2

    B = 128
    assert r0 % B == 0
    grid = r0 // B
    vt = values.T.reshape(r1, r0, 1)

    def c3(x):
        return x.reshape(1, 1, nbins)

    cspec = pl.BlockSpec((1, 1, nbins), lambda i: (0, 0, 0))
    out = pl.pallas_call(
        _twohot_body,
        grid=(grid,),
        in_specs=[
            pl.BlockSpec((r1, B, 1), lambda i: (0, i, 0)),
            cspec, cspec, cspec, cspec, cspec,
        ],
        out_specs=pl.BlockSpec((r1, B, nbins), lambda i: (0, i, 0)),
        out_shape=jax.ShapeDtypeStruct((r1, r0, nbins), jnp.float32),
        compiler_params=pltpu.CompilerParams(
            dimension_semantics=("arbitrary",),
        ),
    )(vt, c3(bins), c3(u1), c3(au1), c3(u2), c3(au2))
    return out.transpose(1, 0, 2)


# speedup vs baseline: 11.4378x; 2.0356x over previous
import functools

import jax
import jax.numpy as jnp
from jax.experimental import pallas as pl
from jax.experimental.pallas import tpu as pltpu


def _twohot_body(v_ref, b_ref, u1_ref, au1_ref, u2_ref, au2_ref, o_ref):
    v = jnp.maximum(v_ref[...], b_ref[0, 0, 0])[:, :, None]
    t1 = jnp.clip(v * u1_ref[...] - au1_ref[...], 0.0, 1.0)
    t2 = jnp.clip(v * u2_ref[...] - au2_ref[...], 0.0, 1.0)
    o_ref[...] = t1 - t2


def kernel(values, bin_values):
    r0, r1 = values.shape
    nbins = bin_values.shape[0]
    bins = bin_values
    u1i = 1.0 / (bins[1:] - bins[:-1])
    u1 = jnp.concatenate([jnp.zeros((1,), jnp.float32), u1i])
    au1 = jnp.concatenate([jnp.full((1,), -1.0, jnp.float32), bins[:-1] * u1i])
    nxt = jnp.concatenate([bins[1:], bins[-1:]])
    d2 = nxt - bins
    u2 = jnp.where(d2 > 0, 1.0 / jnp.maximum(d2, 1e-30), 0.0)
    au2 = bins * u2

    B = 128
    assert r0 % B == 0
    gi = r0 // B
    vt = values.T

    def c3(x):
        return x.reshape(1, 1, nbins)

    cspec = pl.BlockSpec((1, 1, nbins), lambda i: (0, 0, 0))
    out = pl.pallas_call(
        _twohot_body,
        grid=(gi,),
        in_specs=[
            pl.BlockSpec((r1, B), lambda i: (0, i)),
            cspec, cspec, cspec, cspec, cspec,
        ],
        out_specs=pl.BlockSpec((r1, B, nbins), lambda i: (0, i, 0)),
        out_shape=jax.ShapeDtypeStruct((r1, r0, nbins), jnp.float32),
        compiler_params=pltpu.CompilerParams(
            dimension_semantics=("arbitrary",),
        ),
    )(vt, c3(bins), c3(u1), c3(au1), c3(u2), c3(au2))
    return out.transpose(1, 0, 2)


# speedup vs baseline: 12.4650x; 1.0898x over previous
import functools

import jax
import jax.numpy as jnp
from jax.experimental import pallas as pl
from jax.experimental.pallas import tpu as pltpu


def _twohot_body(v_ref, b_ref, u1_ref, au1_ref, u2_ref, au2_ref, o_ref):
    v = jnp.maximum(v_ref[...], b_ref[0, 0, 0])[:, :, None]
    t1 = jnp.clip(v * u1_ref[...] - au1_ref[...], 0.0, 1.0)
    t2 = jnp.clip(v * u2_ref[...] - au2_ref[...], 0.0, 1.0)
    o_ref[...] = t1 - t2


def kernel(values, bin_values):
    r0, r1 = values.shape
    nbins = bin_values.shape[0]
    bins = bin_values
    u1i = 1.0 / (bins[1:] - bins[:-1])
    u1 = jnp.concatenate([jnp.zeros((1,), jnp.float32), u1i])
    au1 = jnp.concatenate([jnp.full((1,), -1.0, jnp.float32), bins[:-1] * u1i])
    nxt = jnp.concatenate([bins[1:], bins[-1:]])
    d2 = nxt - bins
    u2 = jnp.where(d2 > 0, 1.0 / jnp.maximum(d2, 1e-30), 0.0)
    au2 = bins * u2

    B = 256
    assert r0 % B == 0
    gi = r0 // B
    vt = values.T

    def c3(x):
        return x.reshape(1, 1, nbins)

    cspec = pl.BlockSpec((1, 1, nbins), lambda i: (0, 0, 0))
    out = pl.pallas_call(
        _twohot_body,
        grid=(gi,),
        in_specs=[
            pl.BlockSpec((r1, B), lambda i: (0, i)),
            cspec, cspec, cspec, cspec, cspec,
        ],
        out_specs=pl.BlockSpec((r1, B, nbins), lambda i: (0, i, 0)),
        out_shape=jax.ShapeDtypeStruct((r1, r0, nbins), jnp.float32),
        compiler_params=pltpu.CompilerParams(
            dimension_semantics=("arbitrary",),
        ),
    )(vt, c3(bins), c3(u1), c3(au1), c3(u2), c3(au2))
    return out.transpose(1, 0, 2)
